# baseline (speedup 1.0000x reference)
import functools

import jax
import jax.numpy as jnp
from jax import lax
from jax.experimental import pallas as pl
from jax.experimental.pallas import tpu as pltpu
from jax.experimental.pallas import tpu_sc as plsc

CAT_DIMS = (1000, 100000, 10000, 48, 2)
EMB_DIMS = (4, 21, 1, 1, 2)
OFFSETS = (0, 4, 25, 26, 27)
OUT_DIM = 29
BATCH = 16384

NC = 2
NS = 16
L = 16
NW = NC * NS
B_PER_W = BATCH // NW
NBLK = 4
BLK = B_PER_W // NBLK
NCHUNK = B_PER_W // L

REGION = []
_pos = 0
for _d in EMB_DIMS:
    REGION.append(_pos)
    _pos += B_PER_W * _d
N_ELEMS = _pos
N_EROWS = N_ELEMS // BLK


def _body(x_hbm, t0, t1, t2, t3, t4, out_hbm,
          i0, i1, i2, i3, i4, eidx, vals, out_v, sem):
    tables = (t0, t1, t2, t3, t4)
    idxs = (i0, i1, i2, i3, i4)

    wid = lax.axis_index("s") * NC + lax.axis_index("c")
    base = wid * B_PER_W

    for t in range(5):
        pltpu.sync_copy(x_hbm.at[t, pl.ds(wid * NBLK, NBLK), :], idxs[t])

    for t in range(5):
        d = EMB_DIMS[t]
        hi = jnp.int32(CAT_DIMS[t] - 1)
        for k in range(NCHUNK):
            v = idxs[t][k // 8, pl.ds((k % 8) * L, L)]
            v = jnp.minimum(jnp.maximum(v, jnp.int32(0)), hi) * jnp.int32(d)
            for c in range(d):
                p = REGION[t] + c * B_PER_W + k * L
                eidx[p // BLK, pl.ds(p % BLK, L)] = v + jnp.int32(c)

    descs = []
    for t in range(5):
        r0 = REGION[t] // BLK
        nr = (B_PER_W * EMB_DIMS[t]) // BLK
        for j in range(nr):
            descs.append(
                pltpu.async_copy(
                    tables[t].at[eidx.at[r0 + j]],
                    vals.at[r0 + j],
                    sem,
                )
            )
    for dsc in descs:
        dsc.wait()

    for k in range(NCHUNK):
        rows16 = jnp.int32(k * L) + lax.iota(jnp.int32, L)
        for t in range(5):
            d = EMB_DIMS[t]
            for c in range(d):
                p = REGION[t] + c * B_PER_W + k * L
                val = vals[p // BLK, pl.ds(p % BLK, L)]
                col = jnp.full((L,), OFFSETS[t] + c, jnp.int32)
                plsc.store_scatter(out_v, [rows16, col], val)

    pltpu.sync_copy(out_v, out_hbm.at[pl.ds(base, B_PER_W), :])


@jax.jit
def _emb_lookup(x_t, t0, t1, t2, t3, t4):
    mesh = plsc.VectorSubcoreMesh(
        core_axis_name="c", subcore_axis_name="s", num_cores=NC, num_subcores=NS
    )
    scratch = (
        [pltpu.VMEM((NBLK, BLK), jnp.int32) for _ in range(5)]
        + [
            pltpu.VMEM((N_EROWS, BLK), jnp.int32),
            pltpu.VMEM((N_EROWS, BLK), jnp.float32),
            pltpu.VMEM((B_PER_W, OUT_DIM), jnp.float32),
            pltpu.SemaphoreType.DMA,
        ]
    )
    fn = pl.kernel(
        _body,
        out_type=jax.ShapeDtypeStruct((BATCH, OUT_DIM), jnp.float32),
        mesh=mesh,
        scratch_types=scratch,
        compiler_params=pltpu.CompilerParams(
            use_tc_tiling_on_sc=False, needs_layout_passes=False
        ),
    )
    return fn(x_t, t0, t1, t2, t3, t4)


def kernel(x, emb0, emb1, emb2, emb3, emb4):
    x_t = x.T.reshape(5, BATCH // BLK, BLK)
    return _emb_lookup(
        x_t,
        emb0.reshape(-1),
        emb1.reshape(-1),
        emb2.reshape(-1),
        emb3.reshape(-1),
        emb4.reshape(-1),
    )


# speedup vs baseline: 4.0230x; 4.0230x over previous
import functools

import jax
import jax.numpy as jnp
from jax import lax
from jax.experimental import pallas as pl
from jax.experimental.pallas import tpu as pltpu
from jax.experimental.pallas import tpu_sc as plsc

CAT_DIMS = (1000, 100000, 10000, 48, 2)
EMB_DIMS = (4, 21, 1, 1, 2)
OFFSETS = (0, 4, 25, 26, 27)
OUT_DIM = 29
BATCH = 16384

NC = 2
NS = 16
L = 16
NW = NC * NS
B_PER_W = BATCH // NW
NBLK = 4
BLK = B_PER_W // NBLK
NCHUNK = B_PER_W // L

V1 = CAT_DIMS[1]
D1 = EMB_DIMS[1]
K_MAX = (24 * V1) // 21 - 2
LAST8 = V1 - 8


def _body(x_hbm, t1_hbm, s0_hbm, s2_hbm, s3_hbm, s4_hbm, out_hbm,
          i0, i1, i2, i3, i4,
          klist, sbuf,
          win, last8,
          st0, st2, st3, st4,
          out_v, sem, sem2):
    idxs = (i0, i1, i2, i3, i4)
    stages = (st0, None, st2, st3, st4)
    s_hbms = (s0_hbm, None, s2_hbm, s3_hbm, s4_hbm)

    wid = lax.axis_index("s") * NC + lax.axis_index("c")
    base = wid * B_PER_W

    small_descs = [
        pltpu.async_copy(s_hbms[t], stages[t], sem2) for t in (0, 2, 3, 4)
    ]
    small_descs.append(
        pltpu.async_copy(t1_hbm.at[pl.ds(LAST8, 8), :], last8, sem2)
    )

    for t in range(5):
        pltpu.sync_copy(x_hbm.at[t, pl.ds(wid * NBLK, NBLK), :], idxs[t])

    iota2 = jnp.int32(2) * lax.iota(jnp.int32, L)
    for k in range(NCHUNK):
        j, o = k // 8, (k % 8) * L
        i = idxs[1][j, pl.ds(o, L)]
        i = jnp.minimum(jnp.maximum(i, jnp.int32(0)), jnp.int32(V1 - 1))
        kk = jnp.minimum((i * jnp.int32(8)) // jnp.int32(7), jnp.int32(K_MAX))
        sbuf[j, pl.ds(o, L)] = i * jnp.int32(24) - kk * jnp.int32(21)
        m = k // 4
        row_m = jnp.full((L,), m, jnp.int32)
        c0 = jnp.int32(32 * (k % 4)) + iota2
        plsc.store_scatter(klist, [row_m, c0], kk)
        plsc.store_scatter(klist, [row_m, c0 + jnp.int32(1)],
                          kk + jnp.int32(1))

    descs = []
    for m in range(2 * NBLK):
        descs.append(
            pltpu.async_copy(
                t1_hbm.at[klist.at[m]],
                win.at[pl.ds(m * BLK, BLK)], sem))
    for dsc in descs:
        dsc.wait()
    for dsc in small_descs:
        dsc.wait()

    iota18 = jnp.int32(18) * lax.iota(jnp.int32, L)
    for k in range(NCHUNK):
        j, o = k // 8, (k % 8) * L
        rows16 = jnp.int32(k * L) + lax.iota(jnp.int32, L)
        s = sbuf[j, pl.ds(o, L)]
        i1v = idxs[1][j, pl.ds(o, L)]
        psel = i1v == jnp.int32(V1 - 1)

        m = k // 4
        rowsel = rows16 + jnp.int32(64 * m)
        colbase = jnp.int32(18 * ((k % 4) * L)) + iota18
        for c in range(D1):
            t = s + jnp.int32(c)
            val = plsc.load_gather(win, [rowsel, colbase + t])
            patch = plsc.load_gather(
                last8,
                [jnp.full((L,), 7, jnp.int32), jnp.full((L,), c, jnp.int32)],
                mask=psel,
            )
            val = jnp.where(psel, patch, val)
            col = jnp.full((L,), OFFSETS[1] + c, jnp.int32)
            plsc.store_scatter(out_v, [rows16, col], val)

        for t in (0, 2, 3, 4):
            d = EMB_DIMS[t]
            v = idxs[t][j, pl.ds(o, L)]
            v = jnp.minimum(jnp.maximum(v, jnp.int32(0)),
                            jnp.int32(CAT_DIMS[t] - 1)) * jnp.int32(d)
            for c in range(d):
                val = plsc.load_gather(stages[t], [v + jnp.int32(c)])
                col = jnp.full((L,), OFFSETS[t] + c, jnp.int32)
                plsc.store_scatter(out_v, [rows16, col], val)

    pltpu.sync_copy(out_v, out_hbm.at[pl.ds(base, B_PER_W), :])


@jax.jit
def _emb_lookup(x_t, emb1, s0, s2, s3, s4):
    mesh = plsc.VectorSubcoreMesh(
        core_axis_name="c", subcore_axis_name="s", num_cores=NC, num_subcores=NS
    )
    scratch = (
        [pltpu.VMEM((NBLK, BLK), jnp.int32) for _ in range(5)]
        + [
            pltpu.VMEM((2 * NBLK, BLK), jnp.int32),
            pltpu.VMEM((NBLK, BLK), jnp.int32),
            pltpu.VMEM((2 * B_PER_W, D1), jnp.float32),
            pltpu.VMEM((8, D1), jnp.float32),
            pltpu.VMEM((CAT_DIMS[0] * EMB_DIMS[0],), jnp.float32),
            pltpu.VMEM((CAT_DIMS[2] * EMB_DIMS[2],), jnp.float32),
            pltpu.VMEM((CAT_DIMS[3] * EMB_DIMS[3],), jnp.float32),
            pltpu.VMEM((CAT_DIMS[4] * EMB_DIMS[4],), jnp.float32),
            pltpu.VMEM((B_PER_W, OUT_DIM), jnp.float32),
            pltpu.SemaphoreType.DMA,
            pltpu.SemaphoreType.DMA,
        ]
    )
    fn = pl.kernel(
        _body,
        out_type=jax.ShapeDtypeStruct((BATCH, OUT_DIM), jnp.float32),
        mesh=mesh,
        scratch_types=scratch,
        compiler_params=pltpu.CompilerParams(
            use_tc_tiling_on_sc=False, needs_layout_passes=False
        ),
    )
    return fn(x_t, emb1, s0, s2, s3, s4)


def kernel(x, emb0, emb1, emb2, emb3, emb4):
    x_t = x.T.reshape(5, BATCH // BLK, BLK)
    return _emb_lookup(
        x_t,
        emb1,
        emb0.reshape(-1),
        emb2.reshape(-1),
        emb3.reshape(-1),
        emb4.reshape(-1),
    )
